# baseline (device time: 7687 ns/iter reference)
import jax
import jax.numpy as jnp
from jax import lax
from jax.experimental import pallas as pl
from jax.experimental.pallas import tpu as pltpu

N_DEV = 4


def kernel(x):
    m, n = x.shape

    def body(x_hbm, out_hbm, x_vmem, out_vmem, gather_ref,
             local_sems, send_sems, recv_sems):
        my = lax.axis_index("i")

        cp_in = pltpu.make_async_copy(x_hbm, x_vmem, local_sems.at[0])
        cp_in.start()

        barrier_sem = pltpu.get_barrier_semaphore()
        for d in range(1, N_DEV):
            tgt = lax.rem(my + d, N_DEV)
            pl.semaphore_signal(
                barrier_sem, inc=1,
                device_id=(tgt,), device_id_type=pl.DeviceIdType.MESH,
            )
        pl.semaphore_wait(barrier_sem, N_DEV - 1)
        cp_in.wait()

        gather_ref[0, :] = jnp.sum(x_vmem[:, :], axis=0)

        rdmas = []
        for d in range(1, N_DEV):
            tgt = lax.rem(my + d, N_DEV)
            rdma = pltpu.make_async_remote_copy(
                src_ref=gather_ref.at[0],
                dst_ref=gather_ref.at[N_DEV - d],
                send_sem=send_sems.at[d - 1],
                recv_sem=recv_sems.at[N_DEV - d - 1],
                device_id=(tgt,),
                device_id_type=pl.DeviceIdType.MESH,
            )
            rdma.start()
            rdmas.append(rdma)

        xb = x_vmem[:, :].astype(jnp.bfloat16)
        row = lax.broadcasted_iota(jnp.int32, (m, m), 0)
        col = lax.broadcasted_iota(jnp.int32, (m, m), 1)
        tri = (col <= row).astype(jnp.bfloat16)
        local = lax.dot_general(
            tri, xb,
            (((1,), (0,)), ((), ())),
            preferred_element_type=jnp.float32,
        )

        for r in rdmas:
            r.wait()

        s_idx = lax.broadcasted_iota(jnp.int32, (N_DEV, 1), 0)
        dev_idx = lax.rem(my + s_idx, N_DEV)
        mask = jnp.logical_and(s_idx > 0, dev_idx < my).astype(jnp.float32)
        offset = jnp.sum(gather_ref[:, :] * mask, axis=0, keepdims=True)

        out_vmem[:, :] = (local + offset).astype(jnp.bfloat16)
        cp_out = pltpu.make_async_copy(out_vmem, out_hbm, local_sems.at[1])
        cp_out.start()
        cp_out.wait()

    return pl.pallas_call(
        body,
        out_shape=jax.ShapeDtypeStruct((m, n), jnp.bfloat16),
        in_specs=[pl.BlockSpec(memory_space=pl.ANY)],
        out_specs=pl.BlockSpec(memory_space=pl.ANY),
        scratch_shapes=[
            pltpu.VMEM((m, n), jnp.float32),
            pltpu.VMEM((m, n), jnp.bfloat16),
            pltpu.VMEM((N_DEV, n), jnp.float32),
            pltpu.SemaphoreType.DMA((2,)),
            pltpu.SemaphoreType.DMA((N_DEV - 1,)),
            pltpu.SemaphoreType.DMA((N_DEV - 1,)),
        ],
        compiler_params=pltpu.CompilerParams(collective_id=0),
    )(x)


# device time: 7357 ns/iter; 1.0449x vs baseline; 1.0449x over previous
import jax
import jax.numpy as jnp
from jax import lax
from jax.experimental import pallas as pl
from jax.experimental.pallas import tpu as pltpu

N_DEV = 4


def kernel(x):
    m, n = x.shape

    def body(x_hbm, out_hbm, x_vmem, out_vmem, gather_ref,
             local_sems, send_sems, recv_sems):
        my = lax.axis_index("i")
        half = m // 2

        cp_in = pltpu.make_async_copy(x_hbm, x_vmem, local_sems.at[0])
        cp_in.start()

        barrier_sem = pltpu.get_barrier_semaphore()
        for d in range(1, N_DEV):
            tgt = lax.rem(my + d, N_DEV)
            pl.semaphore_signal(
                barrier_sem, inc=1,
                device_id=(tgt,), device_id_type=pl.DeviceIdType.MESH,
            )
        pl.semaphore_wait(barrier_sem, N_DEV - 1)

        row = lax.broadcasted_iota(jnp.int32, (m, m), 0)
        col = lax.broadcasted_iota(jnp.int32, (m, m), 1)
        tri = (col <= row).astype(jnp.bfloat16)

        cp_in.wait()

        gather_ref[0, :] = jnp.sum(x_vmem[:, :], axis=0)

        rdmas = []
        for d in range(1, N_DEV):
            tgt = lax.rem(my + d, N_DEV)
            rdma = pltpu.make_async_remote_copy(
                src_ref=gather_ref.at[0],
                dst_ref=gather_ref.at[N_DEV - d],
                send_sem=send_sems.at[d - 1],
                recv_sem=recv_sems.at[N_DEV - d - 1],
                device_id=(tgt,),
                device_id_type=pl.DeviceIdType.MESH,
            )
            rdma.start()
            rdmas.append(rdma)

        xb = x_vmem[:, :].astype(jnp.bfloat16)
        local = lax.dot_general(
            tri, xb,
            (((1,), (0,)), ((), ())),
            preferred_element_type=jnp.float32,
        )

        for r in rdmas:
            r.wait_recv()

        s_idx = lax.broadcasted_iota(jnp.int32, (N_DEV, 1), 0)
        dev_idx = lax.rem(my + s_idx, N_DEV)
        mask = jnp.logical_and(s_idx > 0, dev_idx < my).astype(jnp.float32)
        offset = jnp.sum(gather_ref[:, :] * mask, axis=0, keepdims=True)

        out_vmem[:half, :] = (local[:half, :] + offset).astype(jnp.bfloat16)
        cp_out0 = pltpu.make_async_copy(
            out_vmem.at[pl.ds(0, half)], out_hbm.at[pl.ds(0, half)],
            local_sems.at[1],
        )
        cp_out0.start()
        out_vmem[half:, :] = (local[half:, :] + offset).astype(jnp.bfloat16)
        cp_out1 = pltpu.make_async_copy(
            out_vmem.at[pl.ds(half, half)], out_hbm.at[pl.ds(half, half)],
            local_sems.at[0],
        )
        cp_out1.start()
        for r in rdmas:
            r.wait_send()
        cp_out0.wait()
        cp_out1.wait()

    return pl.pallas_call(
        body,
        out_shape=jax.ShapeDtypeStruct((m, n), jnp.bfloat16),
        in_specs=[pl.BlockSpec(memory_space=pl.ANY)],
        out_specs=pl.BlockSpec(memory_space=pl.ANY),
        scratch_shapes=[
            pltpu.VMEM((m, n), jnp.float32),
            pltpu.VMEM((m, n), jnp.bfloat16),
            pltpu.VMEM((N_DEV, n), jnp.float32),
            pltpu.SemaphoreType.DMA((2,)),
            pltpu.SemaphoreType.DMA((N_DEV - 1,)),
            pltpu.SemaphoreType.DMA((N_DEV - 1,)),
        ],
        compiler_params=pltpu.CompilerParams(collective_id=0),
    )(x)


# device time: 7331 ns/iter; 1.0486x vs baseline; 1.0035x over previous
import jax
import jax.numpy as jnp
from jax import lax
from jax.experimental import pallas as pl
from jax.experimental.pallas import tpu as pltpu

N_DEV = 4


def kernel(x):
    m, n = x.shape

    def body(x_hbm, out_hbm, x_vmem, out_vmem, gather_ref,
             local_sems, send_sems, recv_sems):
        my = lax.axis_index("i")
        half = m // 2

        cp_in = pltpu.make_async_copy(x_hbm, x_vmem, local_sems.at[0])
        cp_in.start()

        barrier_sem = pltpu.get_barrier_semaphore()
        for d in range(1, N_DEV):
            tgt = lax.rem(my + d, N_DEV)
            pl.semaphore_signal(
                barrier_sem, inc=1,
                device_id=(tgt,), device_id_type=pl.DeviceIdType.MESH,
            )
        pl.semaphore_wait(barrier_sem, N_DEV - 1)

        row = lax.broadcasted_iota(jnp.int32, (m, m), 0)
        col = lax.broadcasted_iota(jnp.int32, (m, m), 1)
        tri = (col <= row).astype(jnp.bfloat16)

        cp_in.wait()

        gather_ref[0, :] = jnp.sum(x_vmem[:, :], axis=0)

        rdmas = []
        for d in range(1, N_DEV):
            tgt = lax.rem(my + d, N_DEV)
            rdma = pltpu.make_async_remote_copy(
                src_ref=gather_ref.at[0],
                dst_ref=gather_ref.at[N_DEV - d],
                send_sem=send_sems.at[d - 1],
                recv_sem=recv_sems.at[N_DEV - d - 1],
                device_id=(tgt,),
                device_id_type=pl.DeviceIdType.MESH,
            )
            rdma.start()
            rdmas.append(rdma)

        xb = x_vmem[:, :].astype(jnp.bfloat16)
        local = lax.dot_general(
            tri, xb,
            (((1,), (0,)), ((), ())),
            preferred_element_type=jnp.float32,
        )

        for r in rdmas:
            r.wait_recv()

        s_idx = lax.broadcasted_iota(jnp.int32, (N_DEV, 1), 0)
        dev_idx = lax.rem(my + s_idx, N_DEV)
        mask = jnp.logical_and(s_idx > 0, dev_idx < my).astype(jnp.float32)
        offset = jnp.sum(gather_ref[:, :] * mask, axis=0, keepdims=True)

        out_vmem[:half, :] = (local[:half, :] + offset).astype(jnp.bfloat16)
        cp_out0 = pltpu.make_async_copy(
            out_vmem.at[pl.ds(0, half)], out_hbm.at[pl.ds(0, half)],
            local_sems.at[1],
        )
        cp_out0.start()
        out_vmem[half:, :] = (local[half:, :] + offset).astype(jnp.bfloat16)
        cp_out1 = pltpu.make_async_copy(
            out_vmem.at[pl.ds(half, half)], out_hbm.at[pl.ds(half, half)],
            local_sems.at[0],
        )
        cp_out1.start()
        for r in rdmas:
            r.wait_send()
        cp_out0.wait()
        cp_out1.wait()

    return pl.pallas_call(
        body,
        out_shape=jax.ShapeDtypeStruct((m, n), jnp.bfloat16),
        in_specs=[pl.BlockSpec(memory_space=pltpu.MemorySpace.HBM)],
        out_specs=pl.BlockSpec(memory_space=pltpu.MemorySpace.HBM),
        scratch_shapes=[
            pltpu.VMEM((m, n), jnp.float32),
            pltpu.VMEM((m, n), jnp.bfloat16),
            pltpu.VMEM((N_DEV, n), jnp.float32),
            pltpu.SemaphoreType.DMA((2,)),
            pltpu.SemaphoreType.DMA((N_DEV - 1,)),
            pltpu.SemaphoreType.DMA((N_DEV - 1,)),
        ],
        compiler_params=pltpu.CompilerParams(collective_id=0),
    )(x)
